# baseline (speedup 1.0000x reference)
import functools

import jax
import jax.numpy as jnp
from jax import lax
from jax.experimental import pallas as pl
from jax.experimental.pallas import tpu as pltpu
from jax.experimental.pallas import tpu_sc as plsc

ALPHA = 0.1
K_STEPS = 10
NC = 2
NS = 16
NW = NC * NS
L = 16
R = 320
NPAD = NW * R
F = 128
FG = F // L
EC = 128
UB = 64

_MESH = plsc.VectorSubcoreMesh(
    core_axis_name="c", subcore_axis_name="s", num_cores=NC, num_subcores=NS
)


def _wid():
    return lax.axis_index("s") * NC + lax.axis_index("c")


def _deg_body(epw, src_hbm, dst_hbm, po_hbm, pi_hbm, src_v, dst_v, dout_v, din_v):
    wid = _wid()
    base = wid * epw
    pltpu.sync_copy(src_hbm.at[pl.ds(base, epw)], src_v.at[pl.ds(0, epw)])
    pltpu.sync_copy(dst_hbm.at[pl.ds(base, epw)], dst_v.at[pl.ds(0, epw)])

    @pl.loop(0, NPAD // L)
    def _zero(i):
        z = jnp.zeros((L,), jnp.float32)
        dout_v[pl.ds(i * L, L)] = z
        din_v[pl.ds(i * L, L)] = z

    one_hot = jnp.where(lax.iota(jnp.int32, L) == 0, 1.0, 0.0).astype(jnp.float32)

    @pl.loop(0, epw)
    def _acc(e):
        s = src_v[pl.ds(e, L)][0]
        d = dst_v[pl.ds(e, L)][0]
        dout_v[pl.ds(s, L)] = dout_v[pl.ds(s, L)] + one_hot
        din_v[pl.ds(d, L)] = din_v[pl.ds(d, L)] + one_hot

    pltpu.sync_copy(dout_v.at[pl.ds(0, NPAD)], po_hbm.at[wid])
    pltpu.sync_copy(din_v.at[pl.ds(0, NPAD)], pi_hbm.at[wid])


def _degrees(src, dst):
    epw = src.shape[0] // NW
    deg = functools.partial(
        pl.kernel,
        out_type=(
            jax.ShapeDtypeStruct((NW, NPAD), jnp.float32),
            jax.ShapeDtypeStruct((NW, NPAD), jnp.float32),
        ),
        mesh=_MESH,
        scratch_types=[
            pltpu.VMEM((epw + L,), jnp.int32),
            pltpu.VMEM((epw + L,), jnp.int32),
            pltpu.VMEM((NPAD + L,), jnp.float32),
            pltpu.VMEM((NPAD + L,), jnp.float32),
        ],
    )(functools.partial(_deg_body, epw))
    return deg(src, dst)


def _fc_body(feat_ref, w_ref, b_ref, out_ref):
    out_ref[...] = (
        jnp.dot(feat_ref[...], w_ref[...], preferred_element_type=jnp.float32)
        + b_ref[...]
    )


def _fc(feat, W, b):
    n, f_in = feat.shape
    f_out = W.shape[1]
    blk = 1024
    return pl.pallas_call(
        _fc_body,
        grid=(n // blk,),
        in_specs=[
            pl.BlockSpec((blk, f_in), lambda i: (i, 0)),
            pl.BlockSpec((f_in, f_out), lambda i: (0, 0)),
            pl.BlockSpec((1, f_out), lambda i: (0, 0)),
        ],
        out_specs=pl.BlockSpec((blk, f_out), lambda i: (i, 0)),
        out_shape=jax.ShapeDtypeStruct((n, f_out), jnp.float32),
    )(feat, W, b.reshape(1, -1))


def _combine_body(po_ref, pi_ref, h0_ref, t_ref, g_ref, sin_ref, nout_ref):
    deg_out = jnp.maximum(jnp.sum(po_ref[...], axis=0, keepdims=True), 1.0)
    deg_in = jnp.maximum(jnp.sum(pi_ref[...], axis=0, keepdims=True), 1.0)
    nout = lax.rsqrt(deg_out)
    sin = (1.0 - ALPHA) * lax.rsqrt(deg_in)
    nout_ref[...] = nout
    sin_ref[...] = sin
    h0 = h0_ref[...]
    t_ref[...] = ALPHA * h0
    g_ref[...] = h0 * nout.reshape(-1, 1)


def _combine(po, pi, h0p):
    blk = 1024
    grid = (NPAD // blk,)
    return pl.pallas_call(
        _combine_body,
        grid=grid,
        in_specs=[
            pl.BlockSpec((NW, blk), lambda i: (0, i)),
            pl.BlockSpec((NW, blk), lambda i: (0, i)),
            pl.BlockSpec((blk, F), lambda i: (i, 0)),
        ],
        out_specs=[
            pl.BlockSpec((blk, F), lambda i: (i, 0)),
            pl.BlockSpec((blk, F), lambda i: (i, 0)),
            pl.BlockSpec((1, blk), lambda i: (0, i)),
            pl.BlockSpec((1, blk), lambda i: (0, i)),
        ],
        out_shape=[
            jax.ShapeDtypeStruct((NPAD, F), jnp.float32),
            jax.ShapeDtypeStruct((NPAD, F), jnp.float32),
            jax.ShapeDtypeStruct((1, NPAD), jnp.float32),
            jax.ShapeDtypeStruct((1, NPAD), jnp.float32),
        ],
    )(po, pi, h0p)


SUP = 1024
NQ = SUP // EC


def _prop_body(
    g_hbm, srcs_hbm, ldst_hbm, meta_hbm, sin_hbm, nout_hbm, t_hbm,
    gout_hbm, hout_hbm,
    meta_v, src_v, ldst_vm, buf0, buf1, agg, tch, hch, gch,
    sin_v, nout_v, sem0, sem1,
):
    wid = _wid()
    base_row = wid * R
    pltpu.sync_copy(meta_hbm.at[wid], meta_v)
    mvec = meta_v[pl.ds(0, L)]
    start = mvec[0]
    end = mvec[1]

    @pl.loop(0, R + 8)
    def _zero(r):
        for j in range(FG):
            agg[r, pl.ds(j * L, L)] = jnp.zeros((L,), jnp.float32)

    c0 = (start // EC) * EC
    nsup = (end - c0 + SUP - 1) // SUP
    bufs = (buf0, buf1)
    sems = (sem0, sem1)

    @pl.loop(0, nsup)
    def _sup(t):
        sb = c0 + t * SUP
        pltpu.sync_copy(srcs_hbm.at[pl.ds(sb, SUP)], src_v)
        pltpu.sync_copy(ldst_hbm.at[pl.ds(sb, SUP)], ldst_vm.at[pl.ds(0, SUP)])
        pltpu.async_copy(g_hbm.at[src_v.at[pl.ds(0, EC)]], buf0, sem0)
        for q in range(NQ):
            cur = bufs[q % 2]
            csem = sems[q % 2]
            if q + 1 < NQ:
                pltpu.async_copy(
                    g_hbm.at[src_v.at[pl.ds((q + 1) * EC, EC)]],
                    bufs[(q + 1) % 2],
                    sems[(q + 1) % 2],
                )
            pltpu.make_async_copy(
                g_hbm.at[src_v.at[pl.ds(q * EC, EC)]], cur, csem
            ).wait()
            qb = sb + q * EC

            @plsc.parallel_loop(0, EC, unroll=8)
            def _acc(e, q=q, qb=qb, cur=cur):
                raw = ldst_vm[pl.ds(q * EC + e, L)][0]
                pos = qb + e
                ok = (pos >= start) & (pos < end)
                row = jnp.where(ok, raw, R)
                vals = [cur[e, pl.ds(j * L, L)] for j in range(FG)]
                for j in range(FG):
                    plsc.addupdate(agg.at[row, pl.ds(j * L, L)], vals[j])

    @pl.loop(0, R // UB)
    def _upd(rb):
        rbase = rb * UB
        g0 = base_row + rbase
        pltpu.sync_copy(t_hbm.at[pl.ds(g0, UB)], tch)
        pltpu.sync_copy(sin_hbm.at[pl.ds(g0, UB)], sin_v.at[pl.ds(0, UB)])
        pltpu.sync_copy(nout_hbm.at[pl.ds(g0, UB)], nout_v.at[pl.ds(0, UB)])

        @pl.loop(0, UB)
        def _row(r):
            s = sin_v[pl.ds(r, L)][0]
            no = nout_v[pl.ds(r, L)][0]
            for j in range(FG):
                a = agg[rbase + r, pl.ds(j * L, L)]
                h = a * s + tch[r, pl.ds(j * L, L)]
                hch[r, pl.ds(j * L, L)] = h
                gch[r, pl.ds(j * L, L)] = h * no

        pltpu.sync_copy(hch, hout_hbm.at[pl.ds(g0, UB)])
        pltpu.sync_copy(gch, gout_hbm.at[pl.ds(g0, UB)])


_prop = pl.kernel(
    _prop_body,
    out_type=(
        jax.ShapeDtypeStruct((NPAD, F), jnp.float32),
        jax.ShapeDtypeStruct((NPAD, F), jnp.float32),
    ),
    mesh=_MESH,
    scratch_types=[
        pltpu.VMEM((L,), jnp.int32),
        pltpu.VMEM((SUP,), jnp.int32),
        pltpu.VMEM((SUP + L,), jnp.int32),
        pltpu.VMEM((EC, F), jnp.float32),
        pltpu.VMEM((EC, F), jnp.float32),
        pltpu.VMEM((R + 8, F), jnp.float32),
        pltpu.VMEM((UB, F), jnp.float32),
        pltpu.VMEM((UB, F), jnp.float32),
        pltpu.VMEM((UB, F), jnp.float32),
        pltpu.VMEM((UB + L,), jnp.float32),
        pltpu.VMEM((UB + L,), jnp.float32),
        pltpu.SemaphoreType.DMA,
        pltpu.SemaphoreType.DMA,
    ],
)


@jax.jit
def _run(feat, edge_index, W, b):
    n = feat.shape[0]
    src = edge_index[0]
    dst = edge_index[1]

    order = jnp.argsort(dst)
    dst_s = dst[order]
    src_s = jnp.pad(src[order], (0, SUP))
    ldst_s = jnp.pad(dst_s % R, (0, SUP))
    offsets = jnp.searchsorted(
        dst_s, (jnp.arange(NW + 1) * R).astype(jnp.int32), side="left"
    ).astype(jnp.int32)
    meta = jnp.zeros((NW, L), jnp.int32)
    meta = meta.at[:, 0].set(offsets[:NW])
    meta = meta.at[:, 1].set(offsets[1:])

    po, pi = _degrees(src, dst)

    feat_p = jnp.pad(feat, ((0, NPAD - n), (0, 0)))
    h0p = _fc(feat_p, W, b)

    t_arr, g, sin2d, nout2d = _combine(po, pi, h0p)
    sin = sin2d.reshape(NPAD)
    nout = nout2d.reshape(NPAD)

    h = h0p
    for _ in range(K_STEPS):
        g, h = _prop(g, src_s, ldst_s, meta, sin, nout, t_arr)
    return h[:n]


def kernel(feat, edge_index, W, b):
    return _run(feat, edge_index, W, b)


# speedup vs baseline: 1.2364x; 1.2364x over previous
import functools

import jax
import jax.numpy as jnp
from jax import lax
from jax.experimental import pallas as pl
from jax.experimental.pallas import tpu as pltpu
from jax.experimental.pallas import tpu_sc as plsc

ALPHA = 0.1
K_STEPS = 10
NC = 2
NS = 16
NW = NC * NS
L = 16
R = 320
NPAD = NW * R
F = 128
FG = F // L
EC = 128
UB = 64

_MESH = plsc.VectorSubcoreMesh(
    core_axis_name="c", subcore_axis_name="s", num_cores=NC, num_subcores=NS
)


def _wid():
    return lax.axis_index("s") * NC + lax.axis_index("c")


def _deg_body(epw, src_hbm, dst_hbm, po_hbm, pi_hbm, src_v, dst_v, dout_v, din_v):
    wid = _wid()
    base = wid * epw
    pltpu.sync_copy(src_hbm.at[pl.ds(base, epw)], src_v.at[pl.ds(0, epw)])
    pltpu.sync_copy(dst_hbm.at[pl.ds(base, epw)], dst_v.at[pl.ds(0, epw)])

    @pl.loop(0, NPAD // L)
    def _zero(i):
        z = jnp.zeros((L,), jnp.float32)
        dout_v[pl.ds(i * L, L)] = z
        din_v[pl.ds(i * L, L)] = z

    one_hot = jnp.where(lax.iota(jnp.int32, L) == 0, 1.0, 0.0).astype(jnp.float32)

    @pl.loop(0, epw)
    def _acc(e):
        s = src_v[pl.ds(e, L)][0]
        d = dst_v[pl.ds(e, L)][0]
        dout_v[pl.ds(s, L)] = dout_v[pl.ds(s, L)] + one_hot
        din_v[pl.ds(d, L)] = din_v[pl.ds(d, L)] + one_hot

    pltpu.sync_copy(dout_v.at[pl.ds(0, NPAD)], po_hbm.at[wid])
    pltpu.sync_copy(din_v.at[pl.ds(0, NPAD)], pi_hbm.at[wid])


def _degrees(src, dst):
    epw = src.shape[0] // NW
    deg = functools.partial(
        pl.kernel,
        out_type=(
            jax.ShapeDtypeStruct((NW, NPAD), jnp.float32),
            jax.ShapeDtypeStruct((NW, NPAD), jnp.float32),
        ),
        mesh=_MESH,
        scratch_types=[
            pltpu.VMEM((epw + L,), jnp.int32),
            pltpu.VMEM((epw + L,), jnp.int32),
            pltpu.VMEM((NPAD + L,), jnp.float32),
            pltpu.VMEM((NPAD + L,), jnp.float32),
        ],
    )(functools.partial(_deg_body, epw))
    return deg(src, dst)


def _fc_body(feat_ref, w_ref, b_ref, out_ref):
    out_ref[...] = (
        jnp.dot(feat_ref[...], w_ref[...], preferred_element_type=jnp.float32)
        + b_ref[...]
    )


def _fc(feat, W, b):
    n, f_in = feat.shape
    f_out = W.shape[1]
    blk = 1024
    return pl.pallas_call(
        _fc_body,
        grid=(n // blk,),
        in_specs=[
            pl.BlockSpec((blk, f_in), lambda i: (i, 0)),
            pl.BlockSpec((f_in, f_out), lambda i: (0, 0)),
            pl.BlockSpec((1, f_out), lambda i: (0, 0)),
        ],
        out_specs=pl.BlockSpec((blk, f_out), lambda i: (i, 0)),
        out_shape=jax.ShapeDtypeStruct((n, f_out), jnp.float32),
    )(feat, W, b.reshape(1, -1))


def _combine_body(po_ref, pi_ref, h0_ref, t_ref, g_ref, sin_ref, nout_ref):
    deg_out = jnp.maximum(jnp.sum(po_ref[...], axis=0, keepdims=True), 1.0)
    deg_in = jnp.maximum(jnp.sum(pi_ref[...], axis=0, keepdims=True), 1.0)
    nout = lax.rsqrt(deg_out)
    sin = (1.0 - ALPHA) * lax.rsqrt(deg_in)
    nout_ref[...] = nout
    sin_ref[...] = sin
    h0 = h0_ref[...]
    t_ref[...] = ALPHA * h0
    g_ref[...] = h0 * nout.reshape(-1, 1)


def _combine(po, pi, h0p):
    blk = 1024
    grid = (NPAD // blk,)
    return pl.pallas_call(
        _combine_body,
        grid=grid,
        in_specs=[
            pl.BlockSpec((NW, blk), lambda i: (0, i)),
            pl.BlockSpec((NW, blk), lambda i: (0, i)),
            pl.BlockSpec((blk, F), lambda i: (i, 0)),
        ],
        out_specs=[
            pl.BlockSpec((blk, F), lambda i: (i, 0)),
            pl.BlockSpec((blk, F), lambda i: (i, 0)),
            pl.BlockSpec((1, blk), lambda i: (0, i)),
            pl.BlockSpec((1, blk), lambda i: (0, i)),
        ],
        out_shape=[
            jax.ShapeDtypeStruct((NPAD, F), jnp.float32),
            jax.ShapeDtypeStruct((NPAD, F), jnp.float32),
            jax.ShapeDtypeStruct((1, NPAD), jnp.float32),
            jax.ShapeDtypeStruct((1, NPAD), jnp.float32),
        ],
    )(po, pi, h0p)


SUP = 1024
NQ = SUP // EC


def _prop_body(
    g_hbm, srcs_hbm, ldst_hbm, meta_hbm, sin_hbm, nout_hbm, t_hbm,
    gout_hbm, hout_hbm,
    meta_v, src_v, ldst_vm, buf0, buf1, agg, tch, hch, gch,
    sin_v, nout_v, sem0, sem1,
):
    wid = _wid()
    base_row = wid * R
    pltpu.sync_copy(meta_hbm.at[wid], meta_v)
    mvec = meta_v[pl.ds(0, L)]
    start = mvec[0]
    end = mvec[1]

    @pl.loop(0, R + 8)
    def _zero(r):
        for j in range(FG):
            agg[r, pl.ds(j * L, L)] = jnp.zeros((L,), jnp.float32)

    c0 = (start // EC) * EC
    nsup = (end - c0 + SUP - 1) // SUP
    bufs = (buf0, buf1)
    sems = (sem0, sem1)

    @pl.loop(0, nsup)
    def _sup(t):
        sb = c0 + t * SUP
        pltpu.sync_copy(srcs_hbm.at[pl.ds(sb, SUP)], src_v)
        pltpu.sync_copy(ldst_hbm.at[pl.ds(sb, SUP)], ldst_vm.at[pl.ds(0, SUP)])
        pltpu.async_copy(g_hbm.at[src_v.at[pl.ds(0, EC)]], buf0, sem0)
        for q in range(NQ):
            cur = bufs[q % 2]
            csem = sems[q % 2]
            if q + 1 < NQ:
                pltpu.async_copy(
                    g_hbm.at[src_v.at[pl.ds((q + 1) * EC, EC)]],
                    bufs[(q + 1) % 2],
                    sems[(q + 1) % 2],
                )
            pltpu.make_async_copy(
                g_hbm.at[src_v.at[pl.ds(q * EC, EC)]], cur, csem
            ).wait()
            qb = sb + q * EC

            @plsc.parallel_loop(0, EC, unroll=8)
            def _acc(e, q=q, qb=qb, cur=cur):
                raw = ldst_vm[pl.ds(q * EC + e, L)][0]
                pos = qb + e
                ok = (pos >= start) & (pos < end)
                row = jnp.where(ok, raw, R)
                packs = [cur[e, pl.ds(j2 * L, L)] for j2 in range(FG // 2)]
                for j2 in range(FG // 2):
                    p = packs[j2]
                    a = lax.bitcast_convert_type(
                        lax.shift_left(p, 16), jnp.float32
                    )
                    b = lax.bitcast_convert_type(
                        lax.shift_left(lax.shift_right_logical(p, 16), 16),
                        jnp.float32,
                    )
                    plsc.addupdate(agg.at[row, pl.ds(j2 * 2 * L, L)], a)
                    plsc.addupdate(agg.at[row, pl.ds(j2 * 2 * L + L, L)], b)

    @pl.loop(0, R // UB)
    def _upd(rb):
        rbase = rb * UB
        g0 = base_row + rbase
        pltpu.sync_copy(t_hbm.at[pl.ds(g0, UB)], tch)
        pltpu.sync_copy(sin_hbm.at[pl.ds(g0, UB)], sin_v.at[pl.ds(0, UB)])
        pltpu.sync_copy(nout_hbm.at[pl.ds(g0, UB)], nout_v.at[pl.ds(0, UB)])

        @pl.loop(0, UB)
        def _row(r):
            s = sin_v[pl.ds(r, L)][0]
            no = nout_v[pl.ds(r, L)][0]
            hv = []
            for j in range(FG):
                a = agg[rbase + r, pl.ds(j * L, L)]
                h = a * s + tch[r, pl.ds(j * L, L)]
                hch[r, pl.ds(j * L, L)] = h
                hv.append(h * no)
            half = jnp.full((L,), 0x8000, jnp.int32)
            for j2 in range(FG // 2):
                ia = lax.bitcast_convert_type(hv[2 * j2], jnp.int32)
                ib = lax.bitcast_convert_type(hv[2 * j2 + 1], jnp.int32)
                alo = lax.shift_right_logical(ia + half, 16)
                bhi = lax.shift_left(
                    lax.shift_right_logical(ib + half, 16), 16
                )
                gch[r, pl.ds(j2 * L, L)] = lax.bitwise_or(bhi, alo)

        pltpu.sync_copy(hch, hout_hbm.at[pl.ds(g0, UB)])
        pltpu.sync_copy(gch, gout_hbm.at[pl.ds(g0, UB)])


_prop = pl.kernel(
    _prop_body,
    out_type=(
        jax.ShapeDtypeStruct((NPAD, F // 2), jnp.int32),
        jax.ShapeDtypeStruct((NPAD, F), jnp.float32),
    ),
    mesh=_MESH,
    scratch_types=[
        pltpu.VMEM((L,), jnp.int32),
        pltpu.VMEM((SUP,), jnp.int32),
        pltpu.VMEM((SUP + L,), jnp.int32),
        pltpu.VMEM((EC, F // 2), jnp.int32),
        pltpu.VMEM((EC, F // 2), jnp.int32),
        pltpu.VMEM((R + 8, F), jnp.float32),
        pltpu.VMEM((UB, F), jnp.float32),
        pltpu.VMEM((UB, F), jnp.float32),
        pltpu.VMEM((UB, F // 2), jnp.int32),
        pltpu.VMEM((UB + L,), jnp.float32),
        pltpu.VMEM((UB + L,), jnp.float32),
        pltpu.SemaphoreType.DMA,
        pltpu.SemaphoreType.DMA,
    ],
    compiler_params=pltpu.CompilerParams(use_tc_tiling_on_sc=False),
)


@jax.jit
def _run(feat, edge_index, W, b):
    n = feat.shape[0]
    src = edge_index[0]
    dst = edge_index[1]

    order = jnp.argsort(dst)
    dst_s = dst[order]
    src_s = jnp.pad(src[order], (0, SUP))
    ldst_s = jnp.pad(dst_s % R, (0, SUP))
    offsets = jnp.searchsorted(
        dst_s, (jnp.arange(NW + 1) * R).astype(jnp.int32), side="left"
    ).astype(jnp.int32)
    meta = jnp.zeros((NW, L), jnp.int32)
    meta = meta.at[:, 0].set(offsets[:NW])
    meta = meta.at[:, 1].set(offsets[1:])

    po, pi = _degrees(src, dst)

    feat_p = jnp.pad(feat, ((0, NPAD - n), (0, 0)))
    h0p = _fc(feat_p, W, b)

    t_arr, g_std, sin2d, nout2d = _combine(po, pi, h0p)
    sin = sin2d.reshape(NPAD)
    nout = nout2d.reshape(NPAD)
    b16 = lax.bitcast_convert_type(g_std.astype(jnp.bfloat16), jnp.uint16)
    pairs = b16.reshape(NPAD, FG // 2, 2, L).transpose(0, 1, 3, 2)
    g = lax.bitcast_convert_type(pairs, jnp.int32).reshape(NPAD, F // 2)

    h = h0p
    for _ in range(K_STEPS):
        g, h = _prop(g, src_s, ldst_s, meta, sin, nout, t_arr)
    return h[:n]


def kernel(feat, edge_index, W, b):
    return _run(feat, edge_index, W, b)


# speedup vs baseline: 1.2821x; 1.0370x over previous
import functools

import jax
import jax.numpy as jnp
from jax import lax
from jax.experimental import pallas as pl
from jax.experimental.pallas import tpu as pltpu
from jax.experimental.pallas import tpu_sc as plsc

ALPHA = 0.1
K_STEPS = 10
NC = 2
NS = 16
NW = NC * NS
L = 16
R = 320
NPAD = NW * R
F = 128
FG = F // L
EC = 128
UB = 64

_MESH = plsc.VectorSubcoreMesh(
    core_axis_name="c", subcore_axis_name="s", num_cores=NC, num_subcores=NS
)


def _wid():
    return lax.axis_index("s") * NC + lax.axis_index("c")


def _deg_body(epw, src_hbm, dst_hbm, po_hbm, pi_hbm, src_v, dst_v, dout_v, din_v):
    wid = _wid()
    base = wid * epw
    pltpu.sync_copy(src_hbm.at[pl.ds(base, epw)], src_v.at[pl.ds(0, epw)])
    pltpu.sync_copy(dst_hbm.at[pl.ds(base, epw)], dst_v.at[pl.ds(0, epw)])

    @pl.loop(0, NPAD // L)
    def _zero(i):
        z = jnp.zeros((L,), jnp.float32)
        dout_v[pl.ds(i * L, L)] = z
        din_v[pl.ds(i * L, L)] = z

    one_hot = jnp.where(lax.iota(jnp.int32, L) == 0, 1.0, 0.0).astype(jnp.float32)

    @plsc.parallel_loop(0, epw, unroll=8)
    def _acc(e):
        s = src_v[pl.ds(e, L)][0]
        d = dst_v[pl.ds(e, L)][0]
        plsc.addupdate(dout_v.at[pl.ds(s, L)], one_hot)
        plsc.addupdate(din_v.at[pl.ds(d, L)], one_hot)

    pltpu.sync_copy(dout_v.at[pl.ds(0, NPAD)], po_hbm.at[wid])
    pltpu.sync_copy(din_v.at[pl.ds(0, NPAD)], pi_hbm.at[wid])


def _degrees(src, dst):
    epw = src.shape[0] // NW
    deg = functools.partial(
        pl.kernel,
        out_type=(
            jax.ShapeDtypeStruct((NW, NPAD), jnp.float32),
            jax.ShapeDtypeStruct((NW, NPAD), jnp.float32),
        ),
        mesh=_MESH,
        scratch_types=[
            pltpu.VMEM((epw + L,), jnp.int32),
            pltpu.VMEM((epw + L,), jnp.int32),
            pltpu.VMEM((NPAD + L,), jnp.float32),
            pltpu.VMEM((NPAD + L,), jnp.float32),
        ],
    )(functools.partial(_deg_body, epw))
    return deg(src, dst)


def _fc_body(feat_ref, w_ref, b_ref, out_ref):
    out_ref[...] = (
        jnp.dot(feat_ref[...], w_ref[...], preferred_element_type=jnp.float32)
        + b_ref[...]
    )


def _fc(feat, W, b):
    n, f_in = feat.shape
    f_out = W.shape[1]
    blk = 1024
    return pl.pallas_call(
        _fc_body,
        grid=(n // blk,),
        in_specs=[
            pl.BlockSpec((blk, f_in), lambda i: (i, 0)),
            pl.BlockSpec((f_in, f_out), lambda i: (0, 0)),
            pl.BlockSpec((1, f_out), lambda i: (0, 0)),
        ],
        out_specs=pl.BlockSpec((blk, f_out), lambda i: (i, 0)),
        out_shape=jax.ShapeDtypeStruct((n, f_out), jnp.float32),
    )(feat, W, b.reshape(1, -1))


def _combine_body(po_ref, pi_ref, h0_ref, t_ref, g_ref, sin_ref, nout_ref):
    deg_out = jnp.maximum(jnp.sum(po_ref[...], axis=0, keepdims=True), 1.0)
    deg_in = jnp.maximum(jnp.sum(pi_ref[...], axis=0, keepdims=True), 1.0)
    nout = lax.rsqrt(deg_out)
    sin = (1.0 - ALPHA) * lax.rsqrt(deg_in)
    nout_ref[...] = nout
    sin_ref[...] = sin
    h0 = h0_ref[...]
    t_ref[...] = ALPHA * h0
    g_ref[...] = h0 * nout.reshape(-1, 1)


def _combine(po, pi, h0p):
    blk = 1024
    grid = (NPAD // blk,)
    return pl.pallas_call(
        _combine_body,
        grid=grid,
        in_specs=[
            pl.BlockSpec((NW, blk), lambda i: (0, i)),
            pl.BlockSpec((NW, blk), lambda i: (0, i)),
            pl.BlockSpec((blk, F), lambda i: (i, 0)),
        ],
        out_specs=[
            pl.BlockSpec((blk, F), lambda i: (i, 0)),
            pl.BlockSpec((blk, F), lambda i: (i, 0)),
            pl.BlockSpec((1, blk), lambda i: (0, i)),
            pl.BlockSpec((1, blk), lambda i: (0, i)),
        ],
        out_shape=[
            jax.ShapeDtypeStruct((NPAD, F), jnp.float32),
            jax.ShapeDtypeStruct((NPAD, F), jnp.float32),
            jax.ShapeDtypeStruct((1, NPAD), jnp.float32),
            jax.ShapeDtypeStruct((1, NPAD), jnp.float32),
        ],
    )(po, pi, h0p)


SUP = 2048
NQ = SUP // EC
NBUF = 3


def _prop_body(
    g_hbm, srcs_hbm, ldst_hbm, meta_hbm, sin_hbm, nout_hbm, t_hbm,
    gout_hbm, hout_hbm,
    meta_v, src_v, ldst_vm, buf0, buf1, buf2, agg, tch, hch, gch,
    sin_v, nout_v, sem0, sem1, sem2,
):
    wid = _wid()
    base_row = wid * R
    pltpu.sync_copy(meta_hbm.at[wid], meta_v)
    mvec = meta_v[pl.ds(0, L)]
    start = mvec[0]
    end = mvec[1]

    @pl.loop(0, R + 8)
    def _zero(r):
        for j in range(FG):
            agg[r, pl.ds(j * L, L)] = jnp.zeros((L,), jnp.float32)

    c0 = (start // EC) * EC
    nsup = (end - c0 + SUP - 1) // SUP
    bufs = (buf0, buf1, buf2)
    sems = (sem0, sem1, sem2)

    @pl.loop(0, nsup)
    def _sup(t):
        sb = c0 + t * SUP
        pltpu.sync_copy(srcs_hbm.at[pl.ds(sb, SUP)], src_v)
        pltpu.sync_copy(ldst_hbm.at[pl.ds(sb, SUP)], ldst_vm.at[pl.ds(0, SUP)])
        for q0 in range(NBUF - 1):
            pltpu.async_copy(
                g_hbm.at[src_v.at[pl.ds(q0 * EC, EC)]], bufs[q0], sems[q0]
            )
        for q in range(NQ):
            cur = bufs[q % NBUF]
            csem = sems[q % NBUF]
            if q + NBUF - 1 < NQ:
                qn = q + NBUF - 1
                pltpu.async_copy(
                    g_hbm.at[src_v.at[pl.ds(qn * EC, EC)]],
                    bufs[qn % NBUF],
                    sems[qn % NBUF],
                )
            pltpu.make_async_copy(
                g_hbm.at[src_v.at[pl.ds(q * EC, EC)]], cur, csem
            ).wait()
            qb = sb + q * EC

            @plsc.parallel_loop(0, EC, unroll=8)
            def _acc(e, q=q, qb=qb, cur=cur):
                raw = ldst_vm[pl.ds(q * EC + e, L)][0]
                pos = qb + e
                ok = (pos >= start) & (pos < end)
                row = jnp.where(ok, raw, R)
                packs = [cur[e, pl.ds(j2 * L, L)] for j2 in range(FG // 2)]
                for j2 in range(FG // 2):
                    p = packs[j2]
                    a = lax.bitcast_convert_type(
                        lax.shift_left(p, 16), jnp.float32
                    )
                    b = lax.bitcast_convert_type(
                        lax.shift_left(lax.shift_right_logical(p, 16), 16),
                        jnp.float32,
                    )
                    plsc.addupdate(agg.at[row, pl.ds(j2 * 2 * L, L)], a)
                    plsc.addupdate(agg.at[row, pl.ds(j2 * 2 * L + L, L)], b)

    @pl.loop(0, R // UB)
    def _upd(rb):
        rbase = rb * UB
        g0 = base_row + rbase
        pltpu.sync_copy(t_hbm.at[pl.ds(g0, UB)], tch)
        pltpu.sync_copy(sin_hbm.at[pl.ds(g0, UB)], sin_v.at[pl.ds(0, UB)])
        pltpu.sync_copy(nout_hbm.at[pl.ds(g0, UB)], nout_v.at[pl.ds(0, UB)])

        @plsc.parallel_loop(0, UB, unroll=2)
        def _row(r):
            s = sin_v[pl.ds(r, L)][0]
            no = nout_v[pl.ds(r, L)][0]
            hv = []
            for j in range(FG):
                a = agg[rbase + r, pl.ds(j * L, L)]
                h = a * s + tch[r, pl.ds(j * L, L)]
                hch[r, pl.ds(j * L, L)] = h
                hv.append(h * no)
            half = jnp.full((L,), 0x8000, jnp.int32)
            for j2 in range(FG // 2):
                ia = lax.bitcast_convert_type(hv[2 * j2], jnp.int32)
                ib = lax.bitcast_convert_type(hv[2 * j2 + 1], jnp.int32)
                alo = lax.shift_right_logical(ia + half, 16)
                bhi = lax.shift_left(
                    lax.shift_right_logical(ib + half, 16), 16
                )
                gch[r, pl.ds(j2 * L, L)] = lax.bitwise_or(bhi, alo)

        pltpu.sync_copy(hch, hout_hbm.at[pl.ds(g0, UB)])
        pltpu.sync_copy(gch, gout_hbm.at[pl.ds(g0, UB)])


_prop = pl.kernel(
    _prop_body,
    out_type=(
        jax.ShapeDtypeStruct((NPAD, F // 2), jnp.int32),
        jax.ShapeDtypeStruct((NPAD, F), jnp.float32),
    ),
    mesh=_MESH,
    scratch_types=[
        pltpu.VMEM((L,), jnp.int32),
        pltpu.VMEM((SUP,), jnp.int32),
        pltpu.VMEM((SUP + L,), jnp.int32),
        pltpu.VMEM((EC, F // 2), jnp.int32),
        pltpu.VMEM((EC, F // 2), jnp.int32),
        pltpu.VMEM((EC, F // 2), jnp.int32),
        pltpu.VMEM((R + 8, F), jnp.float32),
        pltpu.VMEM((UB, F), jnp.float32),
        pltpu.VMEM((UB, F), jnp.float32),
        pltpu.VMEM((UB, F // 2), jnp.int32),
        pltpu.VMEM((UB + L,), jnp.float32),
        pltpu.VMEM((UB + L,), jnp.float32),
        pltpu.SemaphoreType.DMA,
        pltpu.SemaphoreType.DMA,
        pltpu.SemaphoreType.DMA,
    ],
    compiler_params=pltpu.CompilerParams(use_tc_tiling_on_sc=False),
)


@jax.jit
def _run(feat, edge_index, W, b):
    n = feat.shape[0]
    src = edge_index[0]
    dst = edge_index[1]

    order = jnp.argsort(dst)
    dst_s = dst[order]
    src_s = jnp.pad(src[order], (0, SUP))
    ldst_s = jnp.pad(dst_s % R, (0, SUP))
    offsets = jnp.searchsorted(
        dst_s, (jnp.arange(NW + 1) * R).astype(jnp.int32), side="left"
    ).astype(jnp.int32)
    meta = jnp.zeros((NW, L), jnp.int32)
    meta = meta.at[:, 0].set(offsets[:NW])
    meta = meta.at[:, 1].set(offsets[1:])

    po, pi = _degrees(src, dst)

    feat_p = jnp.pad(feat, ((0, NPAD - n), (0, 0)))
    h0p = _fc(feat_p, W, b)

    t_arr, g_std, sin2d, nout2d = _combine(po, pi, h0p)
    sin = sin2d.reshape(NPAD)
    nout = nout2d.reshape(NPAD)
    b16 = lax.bitcast_convert_type(g_std.astype(jnp.bfloat16), jnp.uint16)
    pairs = b16.reshape(NPAD, FG // 2, 2, L).transpose(0, 1, 3, 2)
    g = lax.bitcast_convert_type(pairs, jnp.int32).reshape(NPAD, F // 2)

    h = h0p
    for _ in range(K_STEPS):
        g, h = _prop(g, src_s, ldst_s, meta, sin, nout, t_arr)
    return h[:n]


def kernel(feat, edge_index, W, b):
    return _run(feat, edge_index, W, b)


# speedup vs baseline: 1.2946x; 1.0097x over previous
import functools

import jax
import jax.numpy as jnp
from jax import lax
from jax.experimental import pallas as pl
from jax.experimental.pallas import tpu as pltpu
from jax.experimental.pallas import tpu_sc as plsc

ALPHA = 0.1
K_STEPS = 10
NC = 2
NS = 16
NW = NC * NS
L = 16
R = 320
NPAD = NW * R
F = 128
FG = F // L
EC = 128
UB = 64

_MESH = plsc.VectorSubcoreMesh(
    core_axis_name="c", subcore_axis_name="s", num_cores=NC, num_subcores=NS
)


def _wid():
    return lax.axis_index("s") * NC + lax.axis_index("c")


def _deg_body(epw, src_hbm, dst_hbm, po_hbm, pi_hbm, src_v, dst_v, dout_v, din_v):
    wid = _wid()
    base = wid * epw
    pltpu.sync_copy(src_hbm.at[pl.ds(base, epw)], src_v.at[pl.ds(0, epw)])
    pltpu.sync_copy(dst_hbm.at[pl.ds(base, epw)], dst_v.at[pl.ds(0, epw)])

    @pl.loop(0, NPAD // L)
    def _zero(i):
        z = jnp.zeros((L,), jnp.float32)
        dout_v[pl.ds(i * L, L)] = z
        din_v[pl.ds(i * L, L)] = z

    one_hot = jnp.where(lax.iota(jnp.int32, L) == 0, 1.0, 0.0).astype(jnp.float32)

    @plsc.parallel_loop(0, epw, unroll=8)
    def _acc(e):
        s = src_v[pl.ds(e, L)][0]
        d = dst_v[pl.ds(e, L)][0]
        plsc.addupdate(dout_v.at[pl.ds(s, L)], one_hot)
        plsc.addupdate(din_v.at[pl.ds(d, L)], one_hot)

    pltpu.sync_copy(dout_v.at[pl.ds(0, NPAD)], po_hbm.at[wid])
    pltpu.sync_copy(din_v.at[pl.ds(0, NPAD)], pi_hbm.at[wid])


def _degrees(src, dst):
    epw = src.shape[0] // NW
    deg = functools.partial(
        pl.kernel,
        out_type=(
            jax.ShapeDtypeStruct((NW, NPAD), jnp.float32),
            jax.ShapeDtypeStruct((NW, NPAD), jnp.float32),
        ),
        mesh=_MESH,
        scratch_types=[
            pltpu.VMEM((epw + L,), jnp.int32),
            pltpu.VMEM((epw + L,), jnp.int32),
            pltpu.VMEM((NPAD + L,), jnp.float32),
            pltpu.VMEM((NPAD + L,), jnp.float32),
        ],
    )(functools.partial(_deg_body, epw))
    return deg(src, dst)


def _fc_body(feat_ref, w_ref, b_ref, out_ref):
    out_ref[...] = (
        jnp.dot(feat_ref[...], w_ref[...], preferred_element_type=jnp.float32)
        + b_ref[...]
    )


def _fc(feat, W, b):
    n, f_in = feat.shape
    f_out = W.shape[1]
    blk = 1024
    return pl.pallas_call(
        _fc_body,
        grid=(n // blk,),
        in_specs=[
            pl.BlockSpec((blk, f_in), lambda i: (i, 0)),
            pl.BlockSpec((f_in, f_out), lambda i: (0, 0)),
            pl.BlockSpec((1, f_out), lambda i: (0, 0)),
        ],
        out_specs=pl.BlockSpec((blk, f_out), lambda i: (i, 0)),
        out_shape=jax.ShapeDtypeStruct((n, f_out), jnp.float32),
    )(feat, W, b.reshape(1, -1))


def _combine_body(po_ref, pi_ref, h0_ref, t_ref, g_ref, sin_ref, nout_ref):
    deg_out = jnp.maximum(jnp.sum(po_ref[...], axis=0, keepdims=True), 1.0)
    deg_in = jnp.maximum(jnp.sum(pi_ref[...], axis=0, keepdims=True), 1.0)
    nout = lax.rsqrt(deg_out)
    sin = (1.0 - ALPHA) * lax.rsqrt(deg_in)
    nout_ref[...] = nout
    sin_ref[...] = sin
    h0 = h0_ref[...]
    t_ref[...] = ALPHA * h0
    g_ref[...] = h0 * nout.reshape(-1, 1)


def _combine(po, pi, h0p):
    blk = 1024
    grid = (NPAD // blk,)
    return pl.pallas_call(
        _combine_body,
        grid=grid,
        in_specs=[
            pl.BlockSpec((NW, blk), lambda i: (0, i)),
            pl.BlockSpec((NW, blk), lambda i: (0, i)),
            pl.BlockSpec((blk, F), lambda i: (i, 0)),
        ],
        out_specs=[
            pl.BlockSpec((blk, F), lambda i: (i, 0)),
            pl.BlockSpec((blk, F), lambda i: (i, 0)),
            pl.BlockSpec((1, blk), lambda i: (0, i)),
            pl.BlockSpec((1, blk), lambda i: (0, i)),
        ],
        out_shape=[
            jax.ShapeDtypeStruct((NPAD, F), jnp.float32),
            jax.ShapeDtypeStruct((NPAD, F), jnp.float32),
            jax.ShapeDtypeStruct((1, NPAD), jnp.float32),
            jax.ShapeDtypeStruct((1, NPAD), jnp.float32),
        ],
    )(po, pi, h0p)


SUP = 2048
NQ = SUP // EC
NBUF = 3


def _prop_body(
    g_hbm, srcs_hbm, ldst_hbm, meta_hbm, sin_hbm, nout_hbm, t_hbm,
    gout_hbm, hout_hbm,
    meta_v, src_v, ldst_vm, buf0, buf1, buf2, agg, tch, hch, gch,
    sin_v, nout_v, sem0, sem1, sem2,
):
    wid = _wid()
    base_row = wid * R
    pltpu.sync_copy(meta_hbm.at[wid], meta_v)
    mvec = meta_v[pl.ds(0, L)]
    start = mvec[0]
    end = mvec[1]

    @pl.loop(0, R + 8)
    def _zero(r):
        for j in range(FG):
            agg[r, pl.ds(j * L, L)] = jnp.zeros((L,), jnp.float32)

    c0 = (start // EC) * EC
    nsup = (end - c0 + SUP - 1) // SUP
    bufs = (buf0, buf1, buf2)
    sems = (sem0, sem1, sem2)

    @pl.loop(0, nsup)
    def _sup(t):
        sb = c0 + t * SUP
        pltpu.sync_copy(srcs_hbm.at[pl.ds(sb, SUP)], src_v)
        pltpu.sync_copy(ldst_hbm.at[pl.ds(sb, SUP)], ldst_vm.at[pl.ds(0, SUP)])
        for q0 in range(NBUF - 1):
            pltpu.async_copy(
                g_hbm.at[src_v.at[pl.ds(q0 * EC, EC)]], bufs[q0], sems[q0]
            )
        for q in range(NQ):
            cur = bufs[q % NBUF]
            csem = sems[q % NBUF]
            if q + NBUF - 1 < NQ:
                qn = q + NBUF - 1
                pltpu.async_copy(
                    g_hbm.at[src_v.at[pl.ds(qn * EC, EC)]],
                    bufs[qn % NBUF],
                    sems[qn % NBUF],
                )
            pltpu.make_async_copy(
                g_hbm.at[src_v.at[pl.ds(q * EC, EC)]], cur, csem
            ).wait()
            qb = sb + q * EC

            @plsc.parallel_loop(0, EC, unroll=8)
            def _acc(e, q=q, qb=qb, cur=cur):
                raw = ldst_vm[pl.ds(q * EC + e, L)][0]
                pos = qb + e
                ok = (pos >= start) & (pos < end)
                row = jnp.where(ok, raw, R)
                packs = [cur[e, pl.ds(j2 * L, L)] for j2 in range(FG // 2)]
                for j2 in range(FG // 2):
                    p = packs[j2]
                    a = lax.bitcast_convert_type(
                        lax.shift_left(p, 16), jnp.float32
                    )
                    b = lax.bitcast_convert_type(
                        lax.shift_left(lax.shift_right_logical(p, 16), 16),
                        jnp.float32,
                    )
                    plsc.addupdate(agg.at[row, pl.ds(j2 * 2 * L, L)], a)
                    plsc.addupdate(agg.at[row, pl.ds(j2 * 2 * L + L, L)], b)

    @pl.loop(0, R // UB)
    def _upd(rb):
        rbase = rb * UB
        g0 = base_row + rbase
        pltpu.sync_copy(t_hbm.at[pl.ds(g0, UB)], tch)
        pltpu.sync_copy(sin_hbm.at[pl.ds(g0, UB)], sin_v.at[pl.ds(0, UB)])
        pltpu.sync_copy(nout_hbm.at[pl.ds(g0, UB)], nout_v.at[pl.ds(0, UB)])

        @plsc.parallel_loop(0, UB, unroll=2)
        def _row(r):
            s = sin_v[pl.ds(r, L)][0]
            no = nout_v[pl.ds(r, L)][0]
            hv = []
            for j in range(FG):
                a = agg[rbase + r, pl.ds(j * L, L)]
                h = a * s + tch[r, pl.ds(j * L, L)]
                hch[r, pl.ds(j * L, L)] = h
                hv.append(h * no)
            half = jnp.full((L,), 0x8000, jnp.int32)
            for j2 in range(FG // 2):
                ia = lax.bitcast_convert_type(hv[2 * j2], jnp.int32)
                ib = lax.bitcast_convert_type(hv[2 * j2 + 1], jnp.int32)
                alo = lax.shift_right_logical(ia + half, 16)
                bhi = lax.shift_left(
                    lax.shift_right_logical(ib + half, 16), 16
                )
                gch[r, pl.ds(j2 * L, L)] = lax.bitwise_or(bhi, alo)

        pltpu.sync_copy(hch, hout_hbm.at[pl.ds(g0, UB)])
        pltpu.sync_copy(gch, gout_hbm.at[pl.ds(g0, UB)])


_prop = pl.kernel(
    _prop_body,
    out_type=(
        jax.ShapeDtypeStruct((NPAD, F // 2), jnp.int32),
        jax.ShapeDtypeStruct((NPAD, F), jnp.float32),
    ),
    mesh=_MESH,
    scratch_types=[
        pltpu.VMEM((L,), jnp.int32),
        pltpu.VMEM((SUP,), jnp.int32),
        pltpu.VMEM((SUP + L,), jnp.int32),
        pltpu.VMEM((EC, F // 2), jnp.int32),
        pltpu.VMEM((EC, F // 2), jnp.int32),
        pltpu.VMEM((EC, F // 2), jnp.int32),
        pltpu.VMEM((R + 8, F), jnp.float32),
        pltpu.VMEM((UB, F), jnp.float32),
        pltpu.VMEM((UB, F), jnp.float32),
        pltpu.VMEM((UB, F // 2), jnp.int32),
        pltpu.VMEM((UB + L,), jnp.float32),
        pltpu.VMEM((UB + L,), jnp.float32),
        pltpu.SemaphoreType.DMA,
        pltpu.SemaphoreType.DMA,
        pltpu.SemaphoreType.DMA,
    ],
    compiler_params=pltpu.CompilerParams(use_tc_tiling_on_sc=False),
)


@jax.jit
def _run(feat, edge_index, W, b):
    n = feat.shape[0]
    src = edge_index[0]
    dst = edge_index[1]

    owner = dst // R
    packed = jnp.sort(
        lax.shift_left(owner, 19) | jnp.arange(src.shape[0], dtype=jnp.int32)
    )
    order = packed & ((1 << 19) - 1)
    owner_s = lax.shift_right_logical(packed, 19)
    dst_s = dst[order]
    src_s = jnp.pad(src[order], (0, SUP))
    ldst_s = jnp.pad(dst_s % R, (0, SUP))
    offsets = jnp.searchsorted(
        owner_s, jnp.arange(NW + 1, dtype=jnp.int32), side="left"
    ).astype(jnp.int32)
    meta = jnp.zeros((NW, L), jnp.int32)
    meta = meta.at[:, 0].set(offsets[:NW])
    meta = meta.at[:, 1].set(offsets[1:])

    po, pi = _degrees(src, dst)

    feat_p = jnp.pad(feat, ((0, NPAD - n), (0, 0)))
    h0p = _fc(feat_p, W, b)

    t_arr, g_std, sin2d, nout2d = _combine(po, pi, h0p)
    sin = sin2d.reshape(NPAD)
    nout = nout2d.reshape(NPAD)
    b16 = lax.bitcast_convert_type(g_std.astype(jnp.bfloat16), jnp.uint16)
    pairs = b16.reshape(NPAD, FG // 2, 2, L).transpose(0, 1, 3, 2)
    g = lax.bitcast_convert_type(pairs, jnp.int32).reshape(NPAD, F // 2)

    h = h0p
    for _ in range(K_STEPS):
        g, h = _prop(g, src_s, ldst_s, meta, sin, nout, t_arr)
    return h[:n]


def kernel(feat, edge_index, W, b):
    return _run(feat, edge_index, W, b)


# speedup vs baseline: 1.3470x; 1.0405x over previous
import functools

import jax
import jax.numpy as jnp
from jax import lax
from jax.experimental import pallas as pl
from jax.experimental.pallas import tpu as pltpu
from jax.experimental.pallas import tpu_sc as plsc

ALPHA = 0.1
K_STEPS = 10
NC = 2
NS = 16
NW = NC * NS
L = 16
R = 320
NPAD = NW * R
F = 128
FG = F // L
EC = 128
UB = 64

_MESH = plsc.VectorSubcoreMesh(
    core_axis_name="c", subcore_axis_name="s", num_cores=NC, num_subcores=NS
)


def _wid():
    return lax.axis_index("s") * NC + lax.axis_index("c")


def _deg_body(epw, src_hbm, dst_hbm, po_hbm, pi_hbm, src_v, dst_v, dout_v, din_v):
    wid = _wid()
    base = wid * epw
    pltpu.sync_copy(src_hbm.at[pl.ds(base, epw)], src_v.at[pl.ds(0, epw)])
    pltpu.sync_copy(dst_hbm.at[pl.ds(base, epw)], dst_v.at[pl.ds(0, epw)])

    @pl.loop(0, NPAD // L)
    def _zero(i):
        z = jnp.zeros((L,), jnp.float32)
        dout_v[pl.ds(i * L, L)] = z
        din_v[pl.ds(i * L, L)] = z

    one_hot = jnp.where(lax.iota(jnp.int32, L) == 0, 1.0, 0.0).astype(jnp.float32)

    @plsc.parallel_loop(0, epw, unroll=8)
    def _acc(e):
        s = src_v[pl.ds(e, L)][0]
        d = dst_v[pl.ds(e, L)][0]
        plsc.addupdate(dout_v.at[pl.ds(s, L)], one_hot)
        plsc.addupdate(din_v.at[pl.ds(d, L)], one_hot)

    pltpu.sync_copy(dout_v.at[pl.ds(0, NPAD)], po_hbm.at[wid])
    pltpu.sync_copy(din_v.at[pl.ds(0, NPAD)], pi_hbm.at[wid])


def _degrees(src, dst):
    epw = src.shape[0] // NW
    deg = functools.partial(
        pl.kernel,
        out_type=(
            jax.ShapeDtypeStruct((NW, NPAD), jnp.float32),
            jax.ShapeDtypeStruct((NW, NPAD), jnp.float32),
        ),
        mesh=_MESH,
        scratch_types=[
            pltpu.VMEM((epw + L,), jnp.int32),
            pltpu.VMEM((epw + L,), jnp.int32),
            pltpu.VMEM((NPAD + L,), jnp.float32),
            pltpu.VMEM((NPAD + L,), jnp.float32),
        ],
    )(functools.partial(_deg_body, epw))
    return deg(src, dst)


def _fc_body(feat_ref, w_ref, b_ref, out_ref):
    out_ref[...] = (
        jnp.dot(feat_ref[...], w_ref[...], preferred_element_type=jnp.float32)
        + b_ref[...]
    )


def _fc(feat, W, b):
    n, f_in = feat.shape
    f_out = W.shape[1]
    blk = 1024
    return pl.pallas_call(
        _fc_body,
        grid=(n // blk,),
        in_specs=[
            pl.BlockSpec((blk, f_in), lambda i: (i, 0)),
            pl.BlockSpec((f_in, f_out), lambda i: (0, 0)),
            pl.BlockSpec((1, f_out), lambda i: (0, 0)),
        ],
        out_specs=pl.BlockSpec((blk, f_out), lambda i: (i, 0)),
        out_shape=jax.ShapeDtypeStruct((n, f_out), jnp.float32),
    )(feat, W, b.reshape(1, -1))


def _combine_body(po_ref, pi_ref, h0_ref, t_ref, g_ref, sin_ref, nout_ref):
    deg_out = jnp.maximum(jnp.sum(po_ref[...], axis=0, keepdims=True), 1.0)
    deg_in = jnp.maximum(jnp.sum(pi_ref[...], axis=0, keepdims=True), 1.0)
    nout = lax.rsqrt(deg_out)
    sin = (1.0 - ALPHA) * lax.rsqrt(deg_in)
    nout_ref[...] = nout
    sin_ref[...] = sin
    h0 = h0_ref[...]
    t_ref[...] = ALPHA * h0
    g_ref[...] = h0 * nout.reshape(-1, 1)


def _combine(po, pi, h0p):
    blk = 1024
    grid = (NPAD // blk,)
    return pl.pallas_call(
        _combine_body,
        grid=grid,
        in_specs=[
            pl.BlockSpec((NW, blk), lambda i: (0, i)),
            pl.BlockSpec((NW, blk), lambda i: (0, i)),
            pl.BlockSpec((blk, F), lambda i: (i, 0)),
        ],
        out_specs=[
            pl.BlockSpec((blk, F), lambda i: (i, 0)),
            pl.BlockSpec((blk, F), lambda i: (i, 0)),
            pl.BlockSpec((1, blk), lambda i: (0, i)),
            pl.BlockSpec((1, blk), lambda i: (0, i)),
        ],
        out_shape=[
            jax.ShapeDtypeStruct((NPAD, F), jnp.float32),
            jax.ShapeDtypeStruct((NPAD, F), jnp.float32),
            jax.ShapeDtypeStruct((1, NPAD), jnp.float32),
            jax.ShapeDtypeStruct((1, NPAD), jnp.float32),
        ],
    )(po, pi, h0p)


SUP = 2048
NQ = SUP // EC
NBUF = 3


def _prop_body(
    g0_hbm, srcs_hbm, ldst_hbm, meta_hbm, sin_hbm, nout_hbm, t_hbm,
    hout_hbm, g_hbm,
    meta_v, src_v, ldst_vm, buf0, buf1, buf2, agg, tv, hch, gch,
    sin_v, nout_v, sem0, sem1, sem2, xsem,
):
    cid = lax.axis_index("c")
    sid = lax.axis_index("s")
    wid = sid * NC + cid
    base_row = wid * R
    pltpu.sync_copy(meta_hbm.at[wid], meta_v)
    mvec = meta_v[pl.ds(0, L)]
    start = mvec[0]
    end = mvec[1]

    def _gbar():
        plsc.subcore_barrier()

        @pl.when(sid == 0)
        def _x():
            pltpu.semaphore_signal(xsem, 1, core_index=1 - cid)
            pl.semaphore_wait(xsem, 1)

        plsc.subcore_barrier()

    pltpu.sync_copy(t_hbm.at[pl.ds(base_row, R)], tv)
    pltpu.sync_copy(sin_hbm.at[pl.ds(base_row, R)], sin_v.at[pl.ds(0, R)])
    pltpu.sync_copy(nout_hbm.at[pl.ds(base_row, R)], nout_v.at[pl.ds(0, R)])
    pltpu.sync_copy(g0_hbm.at[pl.ds(base_row, R)], g_hbm.at[pl.ds(base_row, R)])
    _gbar()

    c0 = (start // EC) * EC
    nsup = (end - c0 + SUP - 1) // SUP
    bufs = (buf0, buf1, buf2)
    sems = (sem0, sem1, sem2)

    @pl.loop(0, K_STEPS)
    def _step(k):
        @pl.loop(0, R + 8)
        def _zero(r):
            for j in range(FG):
                agg[r, pl.ds(j * L, L)] = jnp.zeros((L,), jnp.float32)

        @pl.loop(0, nsup)
        def _sup(t):
            sb = c0 + t * SUP
            pltpu.sync_copy(srcs_hbm.at[pl.ds(sb, SUP)], src_v)
            pltpu.sync_copy(
                ldst_hbm.at[pl.ds(sb, SUP)], ldst_vm.at[pl.ds(0, SUP)]
            )
            for q0 in range(NBUF - 1):
                pltpu.async_copy(
                    g_hbm.at[src_v.at[pl.ds(q0 * EC, EC)]], bufs[q0], sems[q0]
                )
            for q in range(NQ):
                cur = bufs[q % NBUF]
                csem = sems[q % NBUF]
                if q + NBUF - 1 < NQ:
                    qn = q + NBUF - 1
                    pltpu.async_copy(
                        g_hbm.at[src_v.at[pl.ds(qn * EC, EC)]],
                        bufs[qn % NBUF],
                        sems[qn % NBUF],
                    )
                pltpu.make_async_copy(
                    g_hbm.at[src_v.at[pl.ds(q * EC, EC)]], cur, csem
                ).wait()
                qb = sb + q * EC

                @plsc.parallel_loop(0, EC, unroll=8)
                def _acc(e, q=q, qb=qb, cur=cur):
                    raw = ldst_vm[pl.ds(q * EC + e, L)][0]
                    pos = qb + e
                    ok = (pos >= start) & (pos < end)
                    row = jnp.where(ok, raw, R)
                    packs = [cur[e, pl.ds(j2 * L, L)] for j2 in range(FG // 2)]
                    for j2 in range(FG // 2):
                        p = packs[j2]
                        a = lax.bitcast_convert_type(
                            lax.shift_left(p, 16), jnp.float32
                        )
                        b = lax.bitcast_convert_type(
                            lax.shift_left(lax.shift_right_logical(p, 16), 16),
                            jnp.float32,
                        )
                        plsc.addupdate(agg.at[row, pl.ds(j2 * 2 * L, L)], a)
                        plsc.addupdate(agg.at[row, pl.ds(j2 * 2 * L + L, L)], b)

        _gbar()

        @pl.loop(0, R // UB)
        def _upd(rb):
            rbase = rb * UB
            gr = base_row + rbase

            @plsc.parallel_loop(0, UB, unroll=2)
            def _row(r):
                s = sin_v[pl.ds(rbase + r, L)][0]
                no = nout_v[pl.ds(rbase + r, L)][0]
                hv = []
                for j in range(FG):
                    a = agg[rbase + r, pl.ds(j * L, L)]
                    h = a * s + tv[rbase + r, pl.ds(j * L, L)]
                    hch[r, pl.ds(j * L, L)] = h
                    hv.append(h * no)
                half = jnp.full((L,), 0x8000, jnp.int32)
                for j2 in range(FG // 2):
                    ia = lax.bitcast_convert_type(hv[2 * j2], jnp.int32)
                    ib = lax.bitcast_convert_type(hv[2 * j2 + 1], jnp.int32)
                    alo = lax.shift_right_logical(ia + half, 16)
                    bhi = lax.shift_left(
                        lax.shift_right_logical(ib + half, 16), 16
                    )
                    gch[r, pl.ds(j2 * L, L)] = lax.bitwise_or(bhi, alo)

            pltpu.sync_copy(gch, g_hbm.at[pl.ds(gr, UB)])

            @pl.when(k == K_STEPS - 1)
            def _wh():
                pltpu.sync_copy(hch, hout_hbm.at[pl.ds(gr, UB)])

        _gbar()


_prop = pl.kernel(
    _prop_body,
    out_type=(
        jax.ShapeDtypeStruct((NPAD, F), jnp.float32),
        jax.ShapeDtypeStruct((NPAD, F // 2), jnp.int32),
    ),
    mesh=_MESH,
    scratch_types=[
        pltpu.VMEM((L,), jnp.int32),
        pltpu.VMEM((SUP,), jnp.int32),
        pltpu.VMEM((SUP + L,), jnp.int32),
        pltpu.VMEM((EC, F // 2), jnp.int32),
        pltpu.VMEM((EC, F // 2), jnp.int32),
        pltpu.VMEM((EC, F // 2), jnp.int32),
        pltpu.VMEM((R + 8, F), jnp.float32),
        pltpu.VMEM((R, F), jnp.float32),
        pltpu.VMEM((UB, F), jnp.float32),
        pltpu.VMEM((UB, F // 2), jnp.int32),
        pltpu.VMEM((R + L,), jnp.float32),
        pltpu.VMEM((R + L,), jnp.float32),
        pltpu.SemaphoreType.DMA,
        pltpu.SemaphoreType.DMA,
        pltpu.SemaphoreType.DMA,
        pltpu.SemaphoreType.REGULAR,
    ],
    compiler_params=pltpu.CompilerParams(use_tc_tiling_on_sc=False),
)


@jax.jit
def _run(feat, edge_index, W, b):
    n = feat.shape[0]
    src = edge_index[0]
    dst = edge_index[1]

    owner = dst // R
    packed = jnp.sort(
        lax.shift_left(owner, 19) | jnp.arange(src.shape[0], dtype=jnp.int32)
    )
    order = packed & ((1 << 19) - 1)
    owner_s = lax.shift_right_logical(packed, 19)
    dst_s = dst[order]
    src_s = jnp.pad(src[order], (0, SUP))
    ldst_s = jnp.pad(dst_s % R, (0, SUP))
    offsets = jnp.searchsorted(
        owner_s, jnp.arange(NW + 1, dtype=jnp.int32), side="left"
    ).astype(jnp.int32)
    meta = jnp.zeros((NW, L), jnp.int32)
    meta = meta.at[:, 0].set(offsets[:NW])
    meta = meta.at[:, 1].set(offsets[1:])

    po, pi = _degrees(src, dst)

    feat_p = jnp.pad(feat, ((0, NPAD - n), (0, 0)))
    h0p = _fc(feat_p, W, b)

    t_arr, g_std, sin2d, nout2d = _combine(po, pi, h0p)
    sin = sin2d.reshape(NPAD)
    nout = nout2d.reshape(NPAD)
    b16 = lax.bitcast_convert_type(g_std.astype(jnp.bfloat16), jnp.uint16)
    pairs = b16.reshape(NPAD, FG // 2, 2, L).transpose(0, 1, 3, 2)
    g = lax.bitcast_convert_type(pairs, jnp.int32).reshape(NPAD, F // 2)

    h, _unused_g = _prop(g, src_s, ldst_s, meta, sin, nout, t_arr)
    return h[:n]


def kernel(feat, edge_index, W, b):
    return _run(feat, edge_index, W, b)
